# baseline (speedup 1.0000x reference)
import functools

import jax
import jax.numpy as jnp
from jax import lax
from jax.experimental import pallas as pl
from jax.experimental.pallas import tpu as pltpu
from jax.experimental.pallas import tpu_sc as plsc


def _gather_kernel(B, F, V, D):
    info = plsc.get_sparse_core_info()
    NC, NS, L = info.num_cores, info.num_subcores, info.num_lanes
    NW = NC * NS
    assert B % NW == 0
    bpw = B // NW

    mesh = plsc.VectorSubcoreMesh(core_axis_name="c", subcore_axis_name="s")

    @functools.partial(
        pl.kernel,
        mesh=mesh,
        out_type=jax.ShapeDtypeStruct((B, F, D), jnp.float32),
        compiler_params=pltpu.CompilerParams(use_tc_tiling_on_sc=False),
        scratch_types=[
            pltpu.VMEM((bpw,), jnp.int32),
            pltpu.VMEM((bpw, D), jnp.float32),
            pltpu.SemaphoreType.DMA,
        ],
    )
    def k(idx_hbm, tab_hbm, out_hbm, ids_v, rows_v, sem):
        wid = lax.axis_index("s") * NC + lax.axis_index("c")
        b0 = wid * bpw

        def field(f, carry):
            pltpu.sync_copy(idx_hbm.at[f, pl.ds(b0, bpw)], ids_v)

            def mkids(i, c):
                sl = pl.ds(i * L, L)
                ids_v[sl] = ids_v[sl] + f * V
                return c

            lax.fori_loop(0, bpw // L, mkids, 0)
            pltpu.async_copy(tab_hbm.at[ids_v], rows_v, sem).wait()
            pltpu.sync_copy(rows_v, out_hbm.at[pl.ds(b0, bpw), f, :])
            return carry

        lax.fori_loop(0, F, field, 0)

    return k


def kernel(inputs, tables):
    B, F = inputs.shape
    _, V, D = tables.shape
    idx_t = inputs.T
    tab_rm = tables.reshape(F * V, D)
    return _gather_kernel(B, F, V, D)(idx_t, tab_rm)
